# Initial kernel scaffold; baseline (speedup 1.0000x reference)
import functools

import jax
import jax.numpy as jnp
from jax import lax
from jax.experimental import pallas as pl
from jax.experimental.pallas import tpu as pltpu
from jax.experimental.pallas import tpu_sc as plsc

B = 16384
D = 64
F = 7
NC = 2
NS = 16
NW = NC * NS
BPW = B // NW
CHUNK = 128
NCH = BPW // CHUNK


def _taxa_body(x_hbm, rows_hbm, e0, e1, e2, e3, e4, e5, e6, out_hbm,
               xv, inds, idxs, acc, gbuf, sem):
    embs = [e0, e1, e2, e3, e4, e5, e6]
    c = lax.axis_index("c")
    s = lax.axis_index("s")
    wid = s * NC + c
    base = wid * BPW

    pltpu.sync_copy(x_hbm.at[pl.ds(wid * NCH, NCH)], xv)

    rdescs = [
        pltpu.async_copy(rows_hbm.at[xv.at[j]],
                         inds.at[pl.ds(j * CHUNK, CHUNK)], sem)
        for j in range(NCH)
    ]
    for d in rdescs:
        d.wait()

    lanes = lax.iota(jnp.int32, 16)
    cols = [jnp.full((16,), f, jnp.int32) for f in range(F)]
    for g in range(BPW // 16):
        ridx = lanes + (g * 16)
        j = g // (CHUNK // 16)
        o = (g % (CHUNK // 16)) * 16
        for f in range(F):
            v = plsc.load_gather(inds, [ridx, cols[f]])
            idxs[f, j, pl.ds(o, 16)] = v

    def gather_field(f, dst):
        return [
            pltpu.async_copy(embs[f].at[idxs.at[f, j]],
                             dst.at[pl.ds(j * CHUNK, CHUNK)], sem)
            for j in range(NCH)
        ]

    for d in gather_field(0, acc):
        d.wait()

    for f in range(1, F):
        for d in gather_field(f, gbuf):
            d.wait()

        @plsc.parallel_loop(0, BPW, unroll=4)
        def _(i):
            for k in range(D // 16):
                plsc.addupdate(acc.at[i, pl.ds(k * 16, 16)],
                               gbuf[i, pl.ds(k * 16, 16)])

    pltpu.sync_copy(acc, out_hbm.at[pl.ds(base, BPW)])


@jax.jit
def _taxa(x2d, rows8, e0, e1, e2, e3, e4, e5, e6):
    mesh = plsc.VectorSubcoreMesh(core_axis_name="c", subcore_axis_name="s")
    return pl.kernel(
        _taxa_body,
        out_type=jax.ShapeDtypeStruct((B, D), jnp.float32),
        mesh=mesh,
        scratch_types=[
            pltpu.VMEM((NCH, CHUNK), jnp.int32),
            pltpu.VMEM((BPW, 8), jnp.int32),
            pltpu.VMEM((F, NCH, CHUNK), jnp.int32),
            pltpu.VMEM((BPW, D), jnp.float32),
            pltpu.VMEM((BPW, D), jnp.float32),
            pltpu.SemaphoreType.DMA,
        ],
    )(x2d, rows8, e0, e1, e2, e3, e4, e5, e6)


def kernel(x, rows, emb0, emb1, emb2, emb3, emb4, emb5, emb6):
    x2d = x.astype(jnp.int32).reshape(NW * NCH, CHUNK)
    rows8 = jnp.pad(rows.astype(jnp.int32), ((0, 0), (0, 1)))
    return _taxa(x2d, rows8, emb0, emb1, emb2, emb3, emb4, emb5, emb6)


# speedup vs baseline: 2.1294x; 2.1294x over previous
import jax
import jax.numpy as jnp
from jax import lax
from jax.experimental import pallas as pl
from jax.experimental.pallas import tpu as pltpu
from jax.experimental.pallas import tpu_sc as plsc

B = 16384
D = 64
F = 7
NC = 2
NS = 16
NW = NC * NS
BPW = B // NW
CHUNK = 128
NCH = BPW // CHUNK


def _taxa_body(x_hbm, c0, c1, c2, c3, c4, c5, c6,
               e0, e1, e2, e3, e4, e5, e6, out_hbm,
               xv, idxs, acc, gbuf, sem):
    cols = [c0, c1, c2, c3, c4, c5, c6]
    embs = [e0, e1, e2, e3, e4, e5, e6]
    c = lax.axis_index("c")
    s = lax.axis_index("s")
    wid = s * NC + c

    pltpu.sync_copy(x_hbm.at[pl.ds(wid * NCH, NCH)], xv)

    idescs = [
        pltpu.async_copy(cols[f].at[xv.at[j]], idxs.at[f, j], sem)
        for f in range(F)
        for j in range(NCH)
    ]
    for d in idescs:
        d.wait()

    def gather_field(f, dst):
        return [
            pltpu.async_copy(embs[f].at[idxs.at[f, j]],
                             dst.at[pl.ds(j * CHUNK, CHUNK)], sem)
            for j in range(NCH)
        ]

    for d in gather_field(0, acc):
        d.wait()

    for f in range(1, F):
        for d in gather_field(f, gbuf):
            d.wait()

        @plsc.parallel_loop(0, BPW, unroll=4)
        def _(i):
            for k in range(D // 16):
                plsc.addupdate(acc.at[i, pl.ds(k * 16, 16)],
                               gbuf[i, pl.ds(k * 16, 16)])

    pltpu.sync_copy(acc, out_hbm.at[pl.ds(wid * BPW, BPW)])


@jax.jit
def _taxa(x2d, cols, embs):
    mesh = plsc.VectorSubcoreMesh(core_axis_name="c", subcore_axis_name="s")
    return pl.kernel(
        _taxa_body,
        out_type=jax.ShapeDtypeStruct((B, D), jnp.float32),
        mesh=mesh,
        scratch_types=[
            pltpu.VMEM((NCH, CHUNK), jnp.int32),
            pltpu.VMEM((F, NCH, CHUNK), jnp.int32),
            pltpu.VMEM((BPW, D), jnp.float32),
            pltpu.VMEM((BPW, D), jnp.float32),
            pltpu.SemaphoreType.DMA,
        ],
        compiler_params=pltpu.CompilerParams(use_tc_tiling_on_sc=False),
    )(x2d, *cols, *embs)


def kernel(x, rows, emb0, emb1, emb2, emb3, emb4, emb5, emb6):
    x2d = x.astype(jnp.int32).reshape(NW * NCH, CHUNK)
    rows32 = rows.astype(jnp.int32)
    cols = [rows32[:, f] for f in range(F)]
    embs = [emb0, emb1, emb2, emb3, emb4, emb5, emb6]
    return _taxa(x2d, cols, embs)
